# baseline (speedup 1.0000x reference)
import functools

import jax
import jax.numpy as jnp
from jax import lax
from jax.experimental import pallas as pl
from jax.experimental.pallas import tpu as pltpu
from jax.experimental.pallas import tpu_sc as plsc

N = 10000
E = 160000
D_IN = 256
D_HID = 512
D_OUT = 256

NC = 2
NT = 16
NQ = 4
FQ = D_IN // NQ
EPT = E // NT
NBUF = 5
K = 128
EP = ((EPT + NBUF * K - 1) // (NBUF * K)) * (NBUF * K)
C = EP // K
NP = 10240
RPT = NP // NT


def _sc_agg_body(inp4_hbm, src_hbm, dst_hbm, w_hbm, zero_hbm, out_hbm,
                 src_v, dst_v, bufs, wbufs, acc, gsems, wsems, ssems):
    c = lax.axis_index("c")
    t = lax.axis_index("s")

    pltpu.sync_copy(dst_hbm.at[t], dst_v)

    def gcopy(j, b):
        return pltpu.make_async_copy(inp4_hbm.at[src_v.at[j]], bufs[b],
                                     gsems[b])

    def wcopy(j, b):
        return pltpu.make_async_copy(w_hbm.at[t, j], wbufs[b], wsems[b])

    def scat_start(j, b):
        pltpu.async_copy(bufs[b], acc.at[dst_v.at[j]], ssems[b], add=True)

    def scat_wait(j, b):
        pltpu.make_async_copy(bufs[b], acc.at[dst_v.at[j]],
                              ssems[b]).wait()

    def fire(j, b):
        gcopy(j, b).start()
        wcopy(j, b).start()

    def wait(j, b):
        gcopy(j, b).wait()
        wcopy(j, b).wait()

    def scale(b):
        buf = bufs[b]
        wbuf = wbufs[b]

        def body(r, carry):
            for q in range(8):
                wv = wbuf[r, pl.ds(q * 16, 16)]
                e = r * 8 + q
                for f in range(FQ // 16):
                    sl = pl.ds(f * 16, 16)
                    buf[e, sl] = buf[e, sl] * wv
            return carry

        lax.fori_loop(0, K // 8, body, 0)

    for p in range(2):
        pltpu.sync_copy(zero_hbm, acc.at[pl.ds(t * RPT, RPT)])
        pltpu.sync_copy(src_hbm.at[c, p, t], src_v)
        plsc.subcore_barrier()

        for b in range(NBUF - 1):
            fire(b, b)

        def loop_body(j4, carry):
            for b in range(NBUF):
                j = NBUF * j4 + b
                wait(j, b)
                scale(b)
                scat_start(j, b)
                jn = j + NBUF - 1
                bn = (b + NBUF - 1) % NBUF

                @pl.when(j >= 1)
                def _():
                    scat_wait(j - 1, bn)

                @pl.when(jn < C)
                def _():
                    fire(jn, bn)
            return carry

        lax.fori_loop(0, C // NBUF, loop_body, 0)

        scat_wait(C - 1, (C - 1) % NBUF)

        plsc.subcore_barrier()
        pltpu.sync_copy(acc.at[pl.ds(t * RPT, RPT)],
                        out_hbm.at[c, p, pl.ds(t * RPT, RPT)])


_sc_agg = functools.partial(
    pl.kernel,
    out_type=jax.ShapeDtypeStruct((NC, 2, NP, FQ), jnp.float32),
    mesh=plsc.VectorSubcoreMesh(core_axis_name="c", subcore_axis_name="s",
                                num_cores=NC, num_subcores=NT),
    scratch_types=[
        pltpu.VMEM((C, K), jnp.int32),
        pltpu.VMEM((C, K), jnp.int32),
        tuple(pltpu.VMEM((K, FQ), jnp.float32)
              for _ in range(NBUF)),
        tuple(pltpu.VMEM((K // 8, 128), jnp.float32)
              for _ in range(NBUF)),
        pltpu.VMEM_SHARED((NP, FQ), jnp.float32),
        tuple(pltpu.SemaphoreType.DMA for _ in range(NBUF)),
        tuple(pltpu.SemaphoreType.DMA for _ in range(NBUF)),
        tuple(pltpu.SemaphoreType.DMA for _ in range(NBUF)),
    ],
    compiler_params=pltpu.CompilerParams(use_tc_tiling_on_sc=False),
)(_sc_agg_body)


BM = 1024


def _tc_dense_body(agg_ref, w4_ref, b_ref, dw_ref, db_ref, out_ref):
    a = agg_ref[...]
    h = b_ref[...]
    for q in range(NQ):
        h = h + jnp.dot(a[q], w4_ref[q],
                        preferred_element_type=jnp.float32)
    h = jnp.where(h > 0, h, jnp.exp(jnp.minimum(h, 0.0)) - 1.0)
    out_ref[...] = (jnp.dot(h, dw_ref[...],
                            preferred_element_type=jnp.float32)
                    + db_ref[...])


def _tc_dense(agg, w4, b, dw, db):
    grid = NP // BM
    return pl.pallas_call(
        _tc_dense_body,
        grid=(grid,),
        in_specs=[
            pl.BlockSpec((NQ, BM, FQ), lambda i: (0, i, 0)),
            pl.BlockSpec((NQ, FQ, D_HID), lambda i: (0, 0, 0)),
            pl.BlockSpec((1, D_HID), lambda i: (0, 0)),
            pl.BlockSpec((D_HID, D_OUT), lambda i: (0, 0)),
            pl.BlockSpec((1, D_OUT), lambda i: (0, 0)),
        ],
        out_specs=pl.BlockSpec((BM, D_OUT), lambda i: (i, 0)),
        out_shape=jax.ShapeDtypeStruct((NP, D_OUT), jnp.float32),
    )(agg, w4, b, dw, db)


def kernel(inp, edge_index, edge_weight, W, b, dense_W, dense_b):
    src = edge_index[0].astype(jnp.int32)
    dst = edge_index[1].astype(jnp.int32)
    w = edge_weight.astype(jnp.float32)

    pad = EP - EPT
    src_p = jnp.pad(src.reshape(NT, EPT), ((0, 0), (0, pad)))
    dst_p = jnp.pad(dst.reshape(NT, EPT), ((0, 0), (0, pad)))
    w_p = jnp.pad(w.reshape(NT, EPT), ((0, 0), (0, pad)))
    src_q = jnp.stack([jnp.stack([src_p + (2 * cc + pp) * N
                                  for pp in range(2)])
                       for cc in range(NC)]).reshape(NC, 2, NT, C, K)
    dst_g = dst_p.reshape(NT, C, K)
    w_g = jnp.broadcast_to(w_p.reshape(NT, C, K, 1),
                           (NT, C, K, 16)).reshape(NT, C, K // 8, 128)

    inp4 = inp.reshape(N, NQ, FQ).transpose(1, 0, 2).reshape(NQ * N, FQ)
    zero = jnp.zeros((RPT, FQ), jnp.float32)

    agg = _sc_agg(inp4, src_q, dst_g, w_g, zero)

    w4 = W.reshape(NQ, FQ, D_HID)
    out = _tc_dense(agg.reshape(NQ, NP, FQ), w4, b.reshape(1, D_HID),
                    dense_W, dense_b.reshape(1, D_OUT))
    return out[:N]


# speedup vs baseline: 1.0738x; 1.0738x over previous
import functools

import jax
import jax.numpy as jnp
from jax import lax
from jax.experimental import pallas as pl
from jax.experimental.pallas import tpu as pltpu
from jax.experimental.pallas import tpu_sc as plsc

N = 10000
E = 160000
D_IN = 256
D_HID = 512
D_OUT = 256

NC = 2
NT = 16
NQ = 4
FQ = D_IN // NQ
EPT = E // NT
NBUF = 5
K = 128
EP = ((EPT + NBUF * K - 1) // (NBUF * K)) * (NBUF * K)
C = EP // K
NP = 10240
RPT = NP // NT


def _sc_agg_body(inp4_hbm, src_hbm, dst_hbm, w_hbm, zero_hbm, out_hbm,
                 src_v, idx_v, dst_v, bufs, wbufs, acc, gsems, wsems, ssems):
    c = lax.axis_index("c")
    t = lax.axis_index("s")

    pltpu.sync_copy(dst_hbm.at[t], dst_v)
    pltpu.sync_copy(src_hbm.at[t], src_v)

    def gcopy(j, b):
        return pltpu.make_async_copy(inp4_hbm.at[idx_v.at[j]], bufs[b],
                                     gsems[b])

    def wcopy(j, b):
        return pltpu.make_async_copy(w_hbm.at[t, j], wbufs[b], wsems[b])

    def scat_start(j, b):
        pltpu.async_copy(bufs[b], acc.at[dst_v.at[j]], ssems[b], add=True)

    def scat_wait(j, b):
        pltpu.make_async_copy(bufs[b], acc.at[dst_v.at[j]],
                              ssems[b]).wait()

    def fire(j, b):
        gcopy(j, b).start()
        wcopy(j, b).start()

    def wait(j, b):
        gcopy(j, b).wait()
        wcopy(j, b).wait()

    def scale(b):
        buf = bufs[b]
        wbuf = wbufs[b]

        def body(r, carry):
            for q in range(8):
                wv = wbuf[r, pl.ds(q * 16, 16)]
                e = r * 8 + q
                for f in range(FQ // 16):
                    sl = pl.ds(f * 16, 16)
                    buf[e, sl] = buf[e, sl] * wv
            return carry

        lax.fori_loop(0, K // 8, body, 0)

    for p in range(2):
        pltpu.sync_copy(zero_hbm, acc.at[pl.ds(t * RPT, RPT)])
        q = 2 * c + p

        def idx_body(i, carry):
            for u in range(K // 16):
                sl = pl.ds(u * 16, 16)
                idx_v[i, sl] = src_v[i, sl] * 4 + q
            return carry

        lax.fori_loop(0, C, idx_body, 0)
        plsc.subcore_barrier()

        for b in range(NBUF - 1):
            fire(b, b)

        def loop_body(j4, carry):
            for b in range(NBUF):
                j = NBUF * j4 + b
                wait(j, b)
                scale(b)
                scat_start(j, b)
                jn = j + NBUF - 1
                bn = (b + NBUF - 1) % NBUF

                @pl.when(j >= 1)
                def _():
                    scat_wait(j - 1, bn)

                @pl.when(jn < C)
                def _():
                    fire(jn, bn)
            return carry

        lax.fori_loop(0, C // NBUF, loop_body, 0)

        scat_wait(C - 1, (C - 1) % NBUF)

        plsc.subcore_barrier()
        pltpu.sync_copy(acc.at[pl.ds(t * RPT, RPT)],
                        out_hbm.at[c, p, pl.ds(t * RPT, RPT)])


_sc_agg = functools.partial(
    pl.kernel,
    out_type=jax.ShapeDtypeStruct((NC, 2, NP, FQ), jnp.float32),
    mesh=plsc.VectorSubcoreMesh(core_axis_name="c", subcore_axis_name="s",
                                num_cores=NC, num_subcores=NT),
    scratch_types=[
        pltpu.VMEM((C, K), jnp.int32),
        pltpu.VMEM((C, K), jnp.int32),
        pltpu.VMEM((C, K), jnp.int32),
        tuple(pltpu.VMEM((K, FQ), jnp.float32)
              for _ in range(NBUF)),
        tuple(pltpu.VMEM((K // 8, 128), jnp.float32)
              for _ in range(NBUF)),
        pltpu.VMEM_SHARED((NP, FQ), jnp.float32),
        tuple(pltpu.SemaphoreType.DMA for _ in range(NBUF)),
        tuple(pltpu.SemaphoreType.DMA for _ in range(NBUF)),
        tuple(pltpu.SemaphoreType.DMA for _ in range(NBUF)),
    ],
    compiler_params=pltpu.CompilerParams(use_tc_tiling_on_sc=False),
)(_sc_agg_body)


BM = 1024


def _tc_dense_body(agg_ref, w4_ref, b_ref, dw_ref, db_ref, out_ref):
    a = agg_ref[...]
    h = b_ref[...]
    for q in range(NQ):
        h = h + jnp.dot(a[q], w4_ref[q],
                        preferred_element_type=jnp.float32)
    h = jnp.where(h > 0, h, jnp.exp(jnp.minimum(h, 0.0)) - 1.0)
    out_ref[...] = (jnp.dot(h, dw_ref[...],
                            preferred_element_type=jnp.float32)
                    + db_ref[...])


def _tc_dense(agg, w4, b, dw, db):
    grid = NP // BM
    return pl.pallas_call(
        _tc_dense_body,
        grid=(grid,),
        in_specs=[
            pl.BlockSpec((NQ, BM, FQ), lambda i: (0, i, 0)),
            pl.BlockSpec((NQ, FQ, D_HID), lambda i: (0, 0, 0)),
            pl.BlockSpec((1, D_HID), lambda i: (0, 0)),
            pl.BlockSpec((D_HID, D_OUT), lambda i: (0, 0)),
            pl.BlockSpec((1, D_OUT), lambda i: (0, 0)),
        ],
        out_specs=pl.BlockSpec((BM, D_OUT), lambda i: (i, 0)),
        out_shape=jax.ShapeDtypeStruct((NP, D_OUT), jnp.float32),
    )(agg, w4, b, dw, db)


def kernel(inp, edge_index, edge_weight, W, b, dense_W, dense_b):
    src = edge_index[0].astype(jnp.int32)
    dst = edge_index[1].astype(jnp.int32)
    w = edge_weight.astype(jnp.float32)

    pad = EP - EPT
    src_p = jnp.pad(src.reshape(NT, EPT), ((0, 0), (0, pad)))
    dst_p = jnp.pad(dst.reshape(NT, EPT), ((0, 0), (0, pad)))
    w_p = jnp.pad(w.reshape(NT, EPT), ((0, 0), (0, pad)))
    src_g = src_p.reshape(NT, C, K)
    dst_g = dst_p.reshape(NT, C, K)
    w_g = jnp.broadcast_to(w_p.reshape(NT, C, K, 1),
                           (NT, C, K, 16)).reshape(NT, C, K // 8, 128)

    inp4 = inp.reshape(NQ * N, FQ)
    zero = jnp.zeros((RPT, FQ), jnp.float32)

    agg = _sc_agg(inp4, src_g, dst_g, w_g, zero)

    w4 = W.reshape(NQ, FQ, D_HID)
    out = _tc_dense(agg.reshape(NQ, NP, FQ), w4, b.reshape(1, D_HID),
                    dense_W, dense_b.reshape(1, D_OUT))
    return out[:N]
